# baseline (speedup 1.0000x reference)
import functools

import jax
import jax.numpy as jnp
from jax import lax
from jax.experimental import pallas as pl
from jax.experimental.pallas import tpu as pltpu
from jax.experimental.pallas import tpu_sc as plsc

N = 10000
E = 320000
D = 128
G = 16
BN_EPS = 1e-5

NC = 2
NS = 16
C = 40
NBUF = 4
EPT = E // NS
CPT = EPT // C
GCH = 100
NGR = CPT // GCH
RPT = 624
RTAIL = N - NS * RPT
PC = 80
PCH = N // PC

_mesh = plsc.VectorSubcoreMesh(core_axis_name="c", subcore_axis_name="s")


@functools.partial(
    pl.kernel,
    out_type=jax.ShapeDtypeStruct((2 * N, D), jnp.float32),
    mesh=_mesh,
    scratch_types=[
        pltpu.VMEM_SHARED((N, D), jnp.float32),
        pltpu.VMEM((GCH, C), jnp.int32),
        pltpu.VMEM((GCH, C), jnp.int32),
        pltpu.VMEM((C, D), jnp.float32),
        pltpu.VMEM((C, D), jnp.float32),
        pltpu.VMEM((C, D), jnp.float32),
        pltpu.VMEM((C, D), jnp.float32),
        pltpu.SemaphoreType.DMA,
        pltpu.SemaphoreType.DMA,
        pltpu.SemaphoreType.DMA,
        pltpu.SemaphoreType.DMA,
        pltpu.SemaphoreType.DMA,
        pltpu.SemaphoreType.DMA,
        pltpu.SemaphoreType.DMA,
        pltpu.SemaphoreType.DMA,
    ],
)
def _gin_aggregate(h2, src2, dst2, out, acc, srcb, dstb,
                   rows0, rows1, rows2, rows3, sem0, sem1, sem2, sem3,
                   ssem0, ssem1, ssem2, ssem3):
    c = lax.axis_index("c")
    s = lax.axis_index("s")
    pltpu.sync_copy(h2.at[pl.ds(c * N + s * RPT, RPT)],
                    acc.at[pl.ds(s * RPT, RPT)])

    @pl.when(s == 0)
    def _():
        pltpu.sync_copy(h2.at[pl.ds(c * N + NS * RPT, RTAIL)],
                        acc.at[pl.ds(NS * RPT, RTAIL)])

    bufs = ((rows0, sem0, ssem0), (rows1, sem1, ssem1),
            (rows2, sem2, ssem2), (rows3, sem3, ssem3))
    pltpu.sync_copy(src2.at[c, s, 0], srcb)
    pltpu.sync_copy(dst2.at[s, 0], dstb)
    plsc.subcore_barrier()

    for g in range(NGR):
        if g > 0:
            pltpu.sync_copy(src2.at[c, s, g], srcb)
            pltpu.sync_copy(dst2.at[s, g], dstb)
        pltpu.make_async_copy(h2.at[srcb.at[0]], rows0, sem0).start()
        pltpu.make_async_copy(h2.at[srcb.at[1]], rows1, sem1).start()

        def body(t, carry):
            for b, (rows, sem, ssem) in enumerate(bufs):
                k = NBUF * t + b
                rows_n, sem_n, ssem_n = bufs[(b + 2) % NBUF]
                pltpu.make_async_copy(h2.at[srcb.at[k]], rows, sem).wait()
                pltpu.async_copy(rows, acc.at[dstb.at[k]], ssem, add=True)

                @pl.when(k >= 2)
                def _():
                    pltpu.make_async_copy(rows_n, acc.at[dstb.at[0]],
                                          ssem_n).wait()

                @pl.when(k + 2 < GCH)
                def _():
                    pltpu.make_async_copy(h2.at[srcb.at[k + 2]],
                                          rows_n, sem_n).start()
            return carry

        lax.fori_loop(0, GCH // NBUF, body, 0)
        for b in ((GCH - 2) % NBUF, (GCH - 1) % NBUF):
            rows, _, ssem = bufs[b]
            pltpu.make_async_copy(rows, acc.at[dstb.at[0]], ssem).wait()
    plsc.subcore_barrier()
    pltpu.sync_copy(acc.at[pl.ds(s * RPT, RPT)],
                    out.at[pl.ds(c * N + s * RPT, RPT)])

    @pl.when(s == 0)
    def _():
        pltpu.sync_copy(acc.at[pl.ds(NS * RPT, RTAIL)],
                        out.at[pl.ds(c * N + NS * RPT, RTAIL)])


@functools.partial(
    pl.kernel,
    out_type=jax.ShapeDtypeStruct((N, D), jnp.float32),
    mesh=_mesh,
    scratch_types=[
        pltpu.VMEM((1, PC), jnp.int32),
        pltpu.VMEM((PC, D), jnp.float32),
        pltpu.SemaphoreType.DMA,
    ],
)
def _permute_rows(x, perm2, xp, idx_v, rows, sem):
    c = lax.axis_index("c")
    s = lax.axis_index("s")
    w = s * NC + c
    for t in range(4):
        j = t * (NC * NS) + w

        @pl.when(j < PCH)
        def _():
            pltpu.sync_copy(perm2.at[j], idx_v)
            cp = pltpu.make_async_copy(x.at[idx_v.at[0]], rows, sem)
            cp.start()
            cp.wait()
            pltpu.sync_copy(rows, xp.at[pl.ds(j * PC, PC)])


def _bn(z, g, b):
    mu = jnp.mean(z, axis=0, keepdims=True)
    var = jnp.mean((z - mu) ** 2, axis=0, keepdims=True)
    return g * (z - mu) / jnp.sqrt(var + BN_EPS) + b


def _mlp_kernel(h_ref, w1_ref, b1_ref, g1_ref, be1_ref, w2_ref, b2_ref,
                g2_ref, be2_ref, out_ref, *, final_relu):
    z = jnp.dot(h_ref[...], w1_ref[...], preferred_element_type=jnp.float32)
    z = z + b1_ref[...]
    g1 = g1_ref[...]
    be1 = be1_ref[...]
    z = jnp.concatenate([_bn(z[:N], g1, be1), _bn(z[N:], g1, be1)], axis=0)
    z = jnp.maximum(z, 0.0)
    z = jnp.dot(z, w2_ref[...], preferred_element_type=jnp.float32)
    z = z + b2_ref[...]
    g2 = g2_ref[...]
    be2 = be2_ref[...]
    z = jnp.concatenate([_bn(z[:N], g2, be2), _bn(z[N:], g2, be2)], axis=0)
    if final_relu:
        z = jnp.maximum(z, 0.0)
    out_ref[...] = z


def _gin_mlp(h, p, i, final_relu):
    return pl.pallas_call(
        functools.partial(_mlp_kernel, final_relu=final_relu),
        out_shape=jax.ShapeDtypeStruct((2 * N, D), jnp.float32),
    )(h, p['l%d_W1' % i], p['l%d_b1' % i][None, :], p['l%d_g1' % i][None, :],
      p['l%d_be1' % i][None, :], p['l%d_W2' % i], p['l%d_b2' % i][None, :],
      p['l%d_g2' % i][None, :], p['l%d_be2' % i][None, :])


def _head_kernel(pos_ref, neg_ref, batch_ref, sw1, sb1, sw2, sb2, dw, db,
                 pos_out, neg_out):
    pos = pos_ref[...]
    neg = neg_ref[...]
    onehot = (batch_ref[...] == lax.broadcasted_iota(jnp.int32, (1, G), 1))
    onehot = onehot.astype(jnp.float32)
    ones = jnp.ones((N, 1), jnp.float32)
    contract = (((0,), (0,)), ((), ()))
    cnt = lax.dot_general(onehot, ones, contract,
                          preferred_element_type=jnp.float32)
    sums = lax.dot_general(onehot, pos, contract,
                           preferred_element_type=jnp.float32)
    summary = sums / jnp.maximum(cnt, 1.0)
    s = jnp.dot(summary, sw1[...], preferred_element_type=jnp.float32)
    s = jnp.maximum(s + sb1[...], 0.0)
    s = jnp.dot(s, sw2[...], preferred_element_type=jnp.float32) + sb2[...]
    s = jax.nn.sigmoid(s)
    sp = jnp.dot(onehot, s, preferred_element_type=jnp.float32)
    dwv = dw[...]
    tpos = jnp.dot(pos, dwv, preferred_element_type=jnp.float32)
    tneg = jnp.dot(neg, dwv, preferred_element_type=jnp.float32)
    pos_out[...] = jnp.sum(tpos * sp, axis=1, keepdims=True) + db[...]
    neg_out[...] = jnp.sum(tneg * sp, axis=1, keepdims=True) + db[...]


def _head(pos, neg, batch, p):
    return pl.pallas_call(
        _head_kernel,
        out_shape=(jax.ShapeDtypeStruct((N, 1), jnp.float32),
                   jax.ShapeDtypeStruct((N, 1), jnp.float32)),
    )(pos, neg, batch.reshape(N, 1),
      p['sum_W1'], p['sum_b1'][None, :], p['sum_W2'], p['sum_b2'][None, :],
      p['disc_W'], p['disc_b'][None, :])


def kernel(x, edge_index, batch, params):
    p = params
    src = edge_index[0]
    dst = edge_index[1]
    perm = jax.random.permutation(jax.random.key(42), N).astype(jnp.int32)
    perm2 = perm.reshape(PCH, 1, PC)
    src2 = jnp.stack([src, src + N]).reshape(2, NS, NGR, GCH, C)
    dst2 = dst.reshape(NS, NGR, GCH, C)

    xp = _permute_rows(x, perm2)
    h = jnp.concatenate([x, xp], axis=0)
    for i in range(3):
        hga = _gin_aggregate(h, src2, dst2)
        h = _gin_mlp(hga, p, i, final_relu=(i != 2))
    ps, ns = _head(h[:N], h[N:], batch, p)
    return ps.reshape(N), ns.reshape(N)


# speedup vs baseline: 1.2958x; 1.2958x over previous
import functools

import jax
import jax.numpy as jnp
from jax import lax
from jax.experimental import pallas as pl
from jax.experimental.pallas import tpu as pltpu
from jax.experimental.pallas import tpu_sc as plsc

N = 10000
E = 320000
D = 128
G = 16
BN_EPS = 1e-5

NC = 2
NS = 16
C = 40
NBUF = 4
EPT = E // NS
CPT = EPT // C
GCH = 100
NGR = CPT // GCH
RPT = 624
RTAIL = N - NS * RPT
PC = 80
PCH = N // PC

_mesh = plsc.VectorSubcoreMesh(core_axis_name="c", subcore_axis_name="s")


@functools.partial(
    pl.kernel,
    out_type=jax.ShapeDtypeStruct((2 * N, D), jnp.float32),
    mesh=_mesh,
    scratch_types=[
        pltpu.VMEM_SHARED((N, D), jnp.float32),
        pltpu.VMEM((GCH, C), jnp.int32),
        pltpu.VMEM((GCH, C), jnp.int32),
        pltpu.VMEM((C, D), jnp.float32),
        pltpu.VMEM((C, D), jnp.float32),
        pltpu.VMEM((C, D), jnp.float32),
        pltpu.VMEM((C, D), jnp.float32),
        pltpu.SemaphoreType.DMA,
        pltpu.SemaphoreType.DMA,
        pltpu.SemaphoreType.DMA,
        pltpu.SemaphoreType.DMA,
        pltpu.SemaphoreType.DMA,
        pltpu.SemaphoreType.DMA,
        pltpu.SemaphoreType.DMA,
        pltpu.SemaphoreType.DMA,
    ],
)
def _gin_aggregate(h2, src2, dst2, out, acc, srcb, dstb,
                   rows0, rows1, rows2, rows3, sem0, sem1, sem2, sem3,
                   ssem0, ssem1, ssem2, ssem3):
    c = lax.axis_index("c")
    s = lax.axis_index("s")
    pltpu.sync_copy(h2.at[pl.ds(c * N + s * RPT, RPT)],
                    acc.at[pl.ds(s * RPT, RPT)])

    @pl.when(s == 0)
    def _():
        pltpu.sync_copy(h2.at[pl.ds(c * N + NS * RPT, RTAIL)],
                        acc.at[pl.ds(NS * RPT, RTAIL)])

    bufs = ((rows0, sem0, ssem0), (rows1, sem1, ssem1),
            (rows2, sem2, ssem2), (rows3, sem3, ssem3))
    pltpu.sync_copy(src2.at[c, s, 0], srcb)
    pltpu.sync_copy(dst2.at[s, 0], dstb)
    plsc.subcore_barrier()

    for g in range(NGR):
        if g > 0:
            pltpu.sync_copy(src2.at[c, s, g], srcb)
            pltpu.sync_copy(dst2.at[s, g], dstb)
        for b, (rows, sem, ssem) in enumerate(bufs):
            pltpu.make_async_copy(h2.at[srcb.at[b]], rows, sem).start()

        def body(t, carry):
            for b, (rows, sem, ssem) in enumerate(bufs):
                k = NBUF * t + b
                pltpu.make_async_copy(h2.at[srcb.at[k]], rows, sem).wait()
                pltpu.sync_copy(rows, acc.at[dstb.at[k]], add=True)

                @pl.when(k + NBUF < GCH)
                def _():
                    pltpu.make_async_copy(h2.at[srcb.at[k + NBUF]],
                                          rows, sem).start()
            return carry

        lax.fori_loop(0, GCH // NBUF, body, 0)
    plsc.subcore_barrier()
    pltpu.sync_copy(acc.at[pl.ds(s * RPT, RPT)],
                    out.at[pl.ds(c * N + s * RPT, RPT)])

    @pl.when(s == 0)
    def _():
        pltpu.sync_copy(acc.at[pl.ds(NS * RPT, RTAIL)],
                        out.at[pl.ds(c * N + NS * RPT, RTAIL)])


@functools.partial(
    pl.kernel,
    out_type=jax.ShapeDtypeStruct((N, D), jnp.float32),
    mesh=_mesh,
    scratch_types=[
        pltpu.VMEM((1, PC), jnp.int32),
        pltpu.VMEM((PC, D), jnp.float32),
        pltpu.SemaphoreType.DMA,
    ],
)
def _permute_rows(x, perm2, xp, idx_v, rows, sem):
    c = lax.axis_index("c")
    s = lax.axis_index("s")
    w = s * NC + c
    for t in range(4):
        j = t * (NC * NS) + w

        @pl.when(j < PCH)
        def _():
            pltpu.sync_copy(perm2.at[j], idx_v)
            cp = pltpu.make_async_copy(x.at[idx_v.at[0]], rows, sem)
            cp.start()
            cp.wait()
            pltpu.sync_copy(rows, xp.at[pl.ds(j * PC, PC)])


def _bn(z, g, b):
    mu = jnp.mean(z, axis=0, keepdims=True)
    var = jnp.mean((z - mu) ** 2, axis=0, keepdims=True)
    return g * (z - mu) / jnp.sqrt(var + BN_EPS) + b


def _mlp_kernel(h_ref, w1_ref, b1_ref, g1_ref, be1_ref, w2_ref, b2_ref,
                g2_ref, be2_ref, out_ref, *, final_relu):
    z = jnp.dot(h_ref[...], w1_ref[...], preferred_element_type=jnp.float32)
    z = z + b1_ref[...]
    g1 = g1_ref[...]
    be1 = be1_ref[...]
    z = jnp.concatenate([_bn(z[:N], g1, be1), _bn(z[N:], g1, be1)], axis=0)
    z = jnp.maximum(z, 0.0)
    z = jnp.dot(z, w2_ref[...], preferred_element_type=jnp.float32)
    z = z + b2_ref[...]
    g2 = g2_ref[...]
    be2 = be2_ref[...]
    z = jnp.concatenate([_bn(z[:N], g2, be2), _bn(z[N:], g2, be2)], axis=0)
    if final_relu:
        z = jnp.maximum(z, 0.0)
    out_ref[...] = z


def _gin_mlp(h, p, i, final_relu):
    return pl.pallas_call(
        functools.partial(_mlp_kernel, final_relu=final_relu),
        out_shape=jax.ShapeDtypeStruct((2 * N, D), jnp.float32),
    )(h, p['l%d_W1' % i], p['l%d_b1' % i][None, :], p['l%d_g1' % i][None, :],
      p['l%d_be1' % i][None, :], p['l%d_W2' % i], p['l%d_b2' % i][None, :],
      p['l%d_g2' % i][None, :], p['l%d_be2' % i][None, :])


def _head_kernel(pos_ref, neg_ref, batch_ref, sw1, sb1, sw2, sb2, dw, db,
                 pos_out, neg_out):
    pos = pos_ref[...]
    neg = neg_ref[...]
    onehot = (batch_ref[...] == lax.broadcasted_iota(jnp.int32, (1, G), 1))
    onehot = onehot.astype(jnp.float32)
    ones = jnp.ones((N, 1), jnp.float32)
    contract = (((0,), (0,)), ((), ()))
    cnt = lax.dot_general(onehot, ones, contract,
                          preferred_element_type=jnp.float32)
    sums = lax.dot_general(onehot, pos, contract,
                           preferred_element_type=jnp.float32)
    summary = sums / jnp.maximum(cnt, 1.0)
    s = jnp.dot(summary, sw1[...], preferred_element_type=jnp.float32)
    s = jnp.maximum(s + sb1[...], 0.0)
    s = jnp.dot(s, sw2[...], preferred_element_type=jnp.float32) + sb2[...]
    s = jax.nn.sigmoid(s)
    sp = jnp.dot(onehot, s, preferred_element_type=jnp.float32)
    dwv = dw[...]
    tpos = jnp.dot(pos, dwv, preferred_element_type=jnp.float32)
    tneg = jnp.dot(neg, dwv, preferred_element_type=jnp.float32)
    pos_out[...] = jnp.sum(tpos * sp, axis=1, keepdims=True) + db[...]
    neg_out[...] = jnp.sum(tneg * sp, axis=1, keepdims=True) + db[...]


def _head(pos, neg, batch, p):
    return pl.pallas_call(
        _head_kernel,
        out_shape=(jax.ShapeDtypeStruct((N, 1), jnp.float32),
                   jax.ShapeDtypeStruct((N, 1), jnp.float32)),
    )(pos, neg, batch.reshape(N, 1),
      p['sum_W1'], p['sum_b1'][None, :], p['sum_W2'], p['sum_b2'][None, :],
      p['disc_W'], p['disc_b'][None, :])


def kernel(x, edge_index, batch, params):
    p = params
    src = edge_index[0]
    dst = edge_index[1]
    perm = jax.random.permutation(jax.random.key(42), N).astype(jnp.int32)
    perm2 = perm.reshape(PCH, 1, PC)
    src2 = jnp.stack([src, src + N]).reshape(2, NS, NGR, GCH, C)
    dst2 = dst.reshape(NS, NGR, GCH, C)

    xp = _permute_rows(x, perm2)
    h = jnp.concatenate([x, xp], axis=0)
    for i in range(3):
        hga = _gin_aggregate(h, src2, dst2)
        h = _gin_mlp(hga, p, i, final_relu=(i != 2))
    ps, ns = _head(h[:N], h[N:], batch, p)
    return ps.reshape(N), ns.reshape(N)


# speedup vs baseline: 1.3389x; 1.0333x over previous
import functools

import jax
import jax.numpy as jnp
from jax import lax
from jax.experimental import pallas as pl
from jax.experimental.pallas import tpu as pltpu
from jax.experimental.pallas import tpu_sc as plsc

N = 10000
E = 320000
D = 128
G = 16
BN_EPS = 1e-5

NC = 2
NS = 16
C = 40
NBUF = 4
EPT = E // NS
CPT = EPT // C
GCH = 100
NGR = CPT // GCH
RPT = 624
RTAIL = N - NS * RPT
PC = 80
PCH = N // PC

_mesh = plsc.VectorSubcoreMesh(core_axis_name="c", subcore_axis_name="s")


@functools.partial(
    pl.kernel,
    out_type=jax.ShapeDtypeStruct((2 * N, D), jnp.float32),
    mesh=_mesh,
    scratch_types=[
        pltpu.VMEM_SHARED((N, D), jnp.float32),
        pltpu.VMEM((GCH, C), jnp.int32),
        pltpu.VMEM((GCH, C), jnp.int32),
        pltpu.VMEM((C, D), jnp.float32),
        pltpu.VMEM((C, D), jnp.float32),
        pltpu.VMEM((C, D), jnp.float32),
        pltpu.VMEM((C, D), jnp.float32),
        pltpu.SemaphoreType.DMA,
        pltpu.SemaphoreType.DMA,
        pltpu.SemaphoreType.DMA,
        pltpu.SemaphoreType.DMA,
        pltpu.SemaphoreType.DMA,
        pltpu.SemaphoreType.DMA,
        pltpu.SemaphoreType.DMA,
        pltpu.SemaphoreType.DMA,
    ],
)
def _gin_aggregate(h2, src2, dst2, out, acc, srcb, dstb,
                   rows0, rows1, rows2, rows3, sem0, sem1, sem2, sem3,
                   ssem0, ssem1, ssem2, ssem3):
    c = lax.axis_index("c")
    s = lax.axis_index("s")
    pltpu.sync_copy(h2.at[pl.ds(c * N + s * RPT, RPT)],
                    acc.at[pl.ds(s * RPT, RPT)])

    @pl.when(s == 0)
    def _():
        pltpu.sync_copy(h2.at[pl.ds(c * N + NS * RPT, RTAIL)],
                        acc.at[pl.ds(NS * RPT, RTAIL)])

    bufs = ((rows0, sem0, ssem0), (rows1, sem1, ssem1),
            (rows2, sem2, ssem2), (rows3, sem3, ssem3))
    pltpu.sync_copy(src2.at[c, s, 0], srcb)
    pltpu.sync_copy(dst2.at[s, 0], dstb)
    plsc.subcore_barrier()

    for g in range(NGR):
        if g > 0:
            pltpu.sync_copy(src2.at[c, s, g], srcb)
            pltpu.sync_copy(dst2.at[s, g], dstb)
        for b, (rows, sem, ssem) in enumerate(bufs):
            pltpu.make_async_copy(h2.at[srcb.at[b]], rows, sem).start()

        def body(t, carry):
            for b, (rows, sem, ssem) in enumerate(bufs):
                k = NBUF * t + b
                pltpu.make_async_copy(h2.at[srcb.at[k]], rows, sem).wait()
                pltpu.sync_copy(rows, acc.at[dstb.at[k]], add=True)

                @pl.when(k + NBUF < GCH)
                def _():
                    pltpu.make_async_copy(h2.at[srcb.at[k + NBUF]],
                                          rows, sem).start()
            return carry

        lax.fori_loop(0, GCH // NBUF, body, 0)
    plsc.subcore_barrier()
    pltpu.sync_copy(acc.at[pl.ds(s * RPT, RPT)],
                    out.at[pl.ds(c * N + s * RPT, RPT)])

    @pl.when(s == 0)
    def _():
        pltpu.sync_copy(acc.at[pl.ds(NS * RPT, RTAIL)],
                        out.at[pl.ds(c * N + NS * RPT, RTAIL)])


@functools.partial(
    pl.kernel,
    out_type=jax.ShapeDtypeStruct((2 * N, D), jnp.float32),
    mesh=_mesh,
    scratch_types=[
        pltpu.VMEM((1, PC), jnp.int32),
        pltpu.VMEM((PC, D), jnp.float32),
        pltpu.SemaphoreType.DMA,
    ],
)
def _build_h0(x, idx2, h0, idx_v, rows, sem):
    c = lax.axis_index("c")
    s = lax.axis_index("s")
    w = s * NC + c
    for t in range(2 * PCH // (NC * NS) + 1):
        j = t * (NC * NS) + w

        @pl.when(j < 2 * PCH)
        def _():
            pltpu.sync_copy(idx2.at[j], idx_v)
            cp = pltpu.make_async_copy(x.at[idx_v.at[0]], rows, sem)
            cp.start()
            cp.wait()
            pltpu.sync_copy(rows, h0.at[pl.ds(j * PC, PC)])


def _bn(z, g, b):
    mu = jnp.mean(z, axis=0, keepdims=True)
    var = jnp.mean((z - mu) ** 2, axis=0, keepdims=True)
    return g * (z - mu) / jnp.sqrt(var + BN_EPS) + b


def _mlp_kernel(h_ref, w1_ref, b1_ref, g1_ref, be1_ref, w2_ref, b2_ref,
                g2_ref, be2_ref, out_ref, *, final_relu):
    z = jnp.dot(h_ref[...], w1_ref[...], preferred_element_type=jnp.float32)
    z = z + b1_ref[...]
    g1 = g1_ref[...]
    be1 = be1_ref[...]
    z = jnp.concatenate([_bn(z[:N], g1, be1), _bn(z[N:], g1, be1)], axis=0)
    z = jnp.maximum(z, 0.0)
    z = jnp.dot(z, w2_ref[...], preferred_element_type=jnp.float32)
    z = z + b2_ref[...]
    g2 = g2_ref[...]
    be2 = be2_ref[...]
    z = jnp.concatenate([_bn(z[:N], g2, be2), _bn(z[N:], g2, be2)], axis=0)
    if final_relu:
        z = jnp.maximum(z, 0.0)
    out_ref[...] = z


def _gin_mlp(h, p, i, final_relu):
    return pl.pallas_call(
        functools.partial(_mlp_kernel, final_relu=final_relu),
        out_shape=jax.ShapeDtypeStruct((2 * N, D), jnp.float32),
    )(h, p['l%d_W1' % i], p['l%d_b1' % i][None, :], p['l%d_g1' % i][None, :],
      p['l%d_be1' % i][None, :], p['l%d_W2' % i], p['l%d_b2' % i][None, :],
      p['l%d_g2' % i][None, :], p['l%d_be2' % i][None, :])


def _mlp_head_kernel(h_ref, w1_ref, b1_ref, g1_ref, be1_ref, w2_ref, b2_ref,
                     g2_ref, be2_ref, batch_ref, sw1, sb1, sw2, sb2, dw, db,
                     pos_out, neg_out):
    z = jnp.dot(h_ref[...], w1_ref[...], preferred_element_type=jnp.float32)
    z = z + b1_ref[...]
    g1 = g1_ref[...]
    be1 = be1_ref[...]
    z = jnp.concatenate([_bn(z[:N], g1, be1), _bn(z[N:], g1, be1)], axis=0)
    z = jnp.maximum(z, 0.0)
    z = jnp.dot(z, w2_ref[...], preferred_element_type=jnp.float32)
    z = z + b2_ref[...]
    g2 = g2_ref[...]
    be2 = be2_ref[...]
    pos = _bn(z[:N], g2, be2)
    neg = _bn(z[N:], g2, be2)
    onehot = (batch_ref[...] == lax.broadcasted_iota(jnp.int32, (1, G), 1))
    onehot = onehot.astype(jnp.float32)
    ones = jnp.ones((N, 1), jnp.float32)
    contract = (((0,), (0,)), ((), ()))
    cnt = lax.dot_general(onehot, ones, contract,
                          preferred_element_type=jnp.float32)
    sums = lax.dot_general(onehot, pos, contract,
                           preferred_element_type=jnp.float32)
    summary = sums / jnp.maximum(cnt, 1.0)
    s = jnp.dot(summary, sw1[...], preferred_element_type=jnp.float32)
    s = jnp.maximum(s + sb1[...], 0.0)
    s = jnp.dot(s, sw2[...], preferred_element_type=jnp.float32) + sb2[...]
    s = jax.nn.sigmoid(s)
    sp = jnp.dot(onehot, s, preferred_element_type=jnp.float32)
    dwv = dw[...]
    tpos = jnp.dot(pos, dwv, preferred_element_type=jnp.float32)
    tneg = jnp.dot(neg, dwv, preferred_element_type=jnp.float32)
    pos_out[...] = jnp.sum(tpos * sp, axis=1, keepdims=True) + db[...]
    neg_out[...] = jnp.sum(tneg * sp, axis=1, keepdims=True) + db[...]


def _mlp_head(h, batch, p):
    i = 2
    return pl.pallas_call(
        _mlp_head_kernel,
        out_shape=(jax.ShapeDtypeStruct((N, 1), jnp.float32),
                   jax.ShapeDtypeStruct((N, 1), jnp.float32)),
    )(h, p['l%d_W1' % i], p['l%d_b1' % i][None, :], p['l%d_g1' % i][None, :],
      p['l%d_be1' % i][None, :], p['l%d_W2' % i], p['l%d_b2' % i][None, :],
      p['l%d_g2' % i][None, :], p['l%d_be2' % i][None, :],
      batch.reshape(N, 1),
      p['sum_W1'], p['sum_b1'][None, :], p['sum_W2'], p['sum_b2'][None, :],
      p['disc_W'], p['disc_b'][None, :])


def kernel(x, edge_index, batch, params):
    p = params
    src = edge_index[0]
    dst = edge_index[1]
    perm = jax.random.permutation(jax.random.key(42), N).astype(jnp.int32)
    idx2 = jnp.concatenate([jnp.arange(N, dtype=jnp.int32), perm])
    idx2 = idx2.reshape(2 * PCH, 1, PC)
    src2 = jnp.stack([src, src + N]).reshape(2, NS, NGR, GCH, C)
    dst2 = dst.reshape(NS, NGR, GCH, C)

    h = _build_h0(x, idx2)
    for i in range(2):
        hga = _gin_aggregate(h, src2, dst2)
        h = _gin_mlp(hga, p, i, final_relu=True)
    hga = _gin_aggregate(h, src2, dst2)
    ps, ns = _mlp_head(hga, batch, p)
    return ps.reshape(N), ns.reshape(N)
